# baseline (speedup 1.0000x reference)
import functools

import jax
import jax.numpy as jnp
from jax import lax
from jax.experimental import pallas as pl
from jax.experimental.pallas import tpu as pltpu
from jax.experimental.pallas import tpu_sc as plsc

N_NODES = 10000
N_EDGES = 320000
D_FEAT = 128
D_HID = 16
N_CLASSES = 7

NC = 2
NS = 16
NW = NC * NS
NPAD = 10240
EPW = 10240
EPAD = EPW * NW
CHUNK = 128
NCHUNK = EPW // CHUNK
NB = 4
ROWS_PT = NPAD // NS


def _mesh():
    return plsc.VectorSubcoreMesh(core_axis_name="c", subcore_axis_name="s")


_SC_PARAMS = pltpu.CompilerParams(use_tc_tiling_on_sc=False,
                                  needs_layout_passes=False)


def _zero_vec(ref, n):
    @pl.loop(0, n // 16)
    def _(i):
        ref[pl.ds(i * 16, 16)] = jnp.zeros((16,), jnp.float32)


def _rsqrt16(x):
    i = plsc.bitcast(x, jnp.int32)
    i = jnp.int32(0x5F3759DF) - lax.shift_right_logical(i, 1)
    y = plsc.bitcast(i, jnp.float32)
    half = x * 0.5
    for _ in range(3):
        y = y * (1.5 - half * y * y)
    return y


@functools.partial(
    pl.kernel,
    out_type=[
        jax.ShapeDtypeStruct((NC, NPAD, D_HID), jnp.float32),
        jax.ShapeDtypeStruct((NPAD,), jnp.float32),
    ],
    mesh=_mesh(),
    compiler_params=_SC_PARAMS,
    scratch_types=[
        pltpu.VMEM((NCHUNK, 2, CHUNK), jnp.int32),
        pltpu.VMEM((NCHUNK, 2, CHUNK), jnp.int32),
        [pltpu.VMEM((CHUNK, D_HID), jnp.float32) for _ in range(NB)],
        pltpu.VMEM((ROWS_PT, D_HID), jnp.float32),
        pltpu.VMEM((ROWS_PT, D_HID), jnp.float32),
        pltpu.VMEM((ROWS_PT,), jnp.float32),
        pltpu.VMEM((ROWS_PT,), jnp.float32),
        pltpu.VMEM((CHUNK,), jnp.float32),
        pltpu.VMEM_SHARED((NPAD,), jnp.float32),
        pltpu.VMEM_SHARED((NPAD, D_HID), jnp.float32),
        pltpu.VMEM_SHARED((NPAD, D_HID), jnp.float32),
        pltpu.SemaphoreType.DMA,
        pltpu.SemaphoreType.DMA,
        [pltpu.SemaphoreType.DMA for _ in range(NB)],
        [pltpu.SemaphoreType.DMA for _ in range(NB)],
    ],
)
def _sc_fused1(h_hbm, ei_hbm, out_hbm, norm_hbm,
               idxb, idxm, rowsb, hb, zb, zflat, normt, onesb,
               degsh, gsh, aggsh, hsem, dsem, gsem, ssem):
    c = lax.axis_index("c")
    s = lax.axis_index("s")
    wid = c * NS + s
    mwid = (1 - c) * NS + s
    row0 = s * ROWS_PT

    hdesc = pltpu.async_copy(h_hbm.at[pl.ds(row0, ROWS_PT)], hb, hsem)
    pltpu.sync_copy(ei_hbm.at[wid], idxb)
    pltpu.sync_copy(ei_hbm.at[mwid], idxm)

    _zero_vec(zflat, ROWS_PT)
    pltpu.sync_copy(zflat, degsh.at[pl.ds(row0, ROWS_PT)])

    @pl.loop(0, ROWS_PT)
    def _(i):
        zb[i, :] = jnp.zeros((D_HID,), jnp.float32)

    pltpu.sync_copy(zb, aggsh.at[pl.ds(row0, ROWS_PT)])

    @pl.loop(0, CHUNK // 16)
    def _(i):
        onesb[pl.ds(i * 16, 16)] = jnp.ones((16,), jnp.float32)

    plsc.subcore_barrier()

    for idxref in (idxb, idxm):
        @pl.loop(0, NCHUNK, step=8)
        def _(j0):
            @pl.loop(0, 8)
            def _(b):
                pltpu.async_copy(onesb, degsh.at[idxref.at[j0 + b, 1]],
                                 dsem, add=True)

            @pl.loop(0, 8)
            def _(b):
                pltpu.make_async_copy(onesb, degsh.at[idxref.at[0, 1]],
                                      dsem).wait()

    plsc.subcore_barrier()

    pltpu.sync_copy(degsh.at[pl.ds(row0, ROWS_PT)], normt)

    @pl.loop(0, ROWS_PT // 16)
    def _(i):
        d16 = jnp.maximum(normt[pl.ds(i * 16, 16)], 1.0)
        normt[pl.ds(i * 16, 16)] = _rsqrt16(d16)

    @pl.when(c == 0)
    def _():
        pltpu.sync_copy(normt, norm_hbm.at[pl.ds(row0, ROWS_PT)])

    hdesc.wait()

    @pl.loop(0, ROWS_PT // 16)
    def _(ii):
        i0 = ii * 16
        n16 = normt[pl.ds(i0, 16)]
        for j in range(16):
            hb[i0 + j, :] = hb[i0 + j, :] * n16[j]

    pltpu.sync_copy(hb, gsh.at[pl.ds(row0, ROWS_PT)])
    plsc.subcore_barrier()

    def start_gather(j, b):
        pltpu.async_copy(gsh.at[idxb.at[j, 0]], rowsb[b], gsem[b])

    def wait_gather(b):
        pltpu.make_async_copy(gsh.at[idxb.at[0, 0]], rowsb[b],
                              gsem[b]).wait()

    def start_scatter(j, b):
        pltpu.async_copy(rowsb[b], aggsh.at[idxb.at[j, 1]], ssem[b],
                         add=True)

    def wait_scatter(b):
        pltpu.make_async_copy(rowsb[b], aggsh.at[idxb.at[0, 1]],
                              ssem[b]).wait()

    for b in range(NB):
        start_gather(b, b)

    @pl.loop(0, (NCHUNK - NB) // NB)
    def _(gg):
        base = gg * NB
        for b in range(NB):
            j = base + b
            wait_gather(b)
            start_scatter(j, b)
            wait_scatter(b)
            start_gather(j + NB, b)

    for b in range(NB):
        j = NCHUNK - NB + b
        wait_gather(b)
        start_scatter(j, b)
        wait_scatter(b)

    plsc.subcore_barrier()
    pltpu.sync_copy(aggsh.at[pl.ds(row0, ROWS_PT)],
                    out_hbm.at[c, pl.ds(row0, ROWS_PT)])


@functools.partial(
    pl.kernel,
    out_type=jax.ShapeDtypeStruct((NC, NPAD, D_HID), jnp.float32),
    mesh=_mesh(),
    compiler_params=_SC_PARAMS,
    scratch_types=[
        pltpu.VMEM((NCHUNK, 2, CHUNK), jnp.int32),
        [pltpu.VMEM((CHUNK, D_HID), jnp.float32) for _ in range(NB)],
        pltpu.VMEM((ROWS_PT, D_HID), jnp.float32),
        pltpu.VMEM_SHARED((NPAD, D_HID), jnp.float32),
        pltpu.VMEM_SHARED((NPAD, D_HID), jnp.float32),
        [pltpu.SemaphoreType.DMA for _ in range(NB)],
        [pltpu.SemaphoreType.DMA for _ in range(NB)],
    ],
)
def _sc_agg2(g_hbm, ei_hbm, out_hbm, idxb, rowsb, zb, aggsh, gsh, gsem, ssem):
    c = lax.axis_index("c")
    s = lax.axis_index("s")
    wid = c * NS + s
    row0 = s * ROWS_PT

    @pl.loop(0, ROWS_PT)
    def _(i):
        zb[i, :] = jnp.zeros((D_HID,), jnp.float32)

    pltpu.sync_copy(zb, aggsh.at[pl.ds(row0, ROWS_PT)])
    pltpu.sync_copy(g_hbm.at[pl.ds(row0, ROWS_PT)],
                    gsh.at[pl.ds(row0, ROWS_PT)])
    pltpu.sync_copy(ei_hbm.at[wid], idxb)
    plsc.subcore_barrier()

    def start_gather(j, b):
        pltpu.async_copy(gsh.at[idxb.at[j, 0]], rowsb[b], gsem[b])

    def wait_gather(b):
        pltpu.make_async_copy(gsh.at[idxb.at[0, 0]], rowsb[b],
                              gsem[b]).wait()

    def start_scatter(j, b):
        pltpu.async_copy(rowsb[b], aggsh.at[idxb.at[j, 1]], ssem[b],
                         add=True)

    def wait_scatter(b):
        pltpu.make_async_copy(rowsb[b], aggsh.at[idxb.at[0, 1]],
                              ssem[b]).wait()

    for b in range(NB):
        start_gather(b, b)

    @pl.loop(0, (NCHUNK - NB) // NB)
    def _(gg):
        base = gg * NB
        for b in range(NB):
            j = base + b
            wait_gather(b)
            start_scatter(j, b)
            wait_scatter(b)
            start_gather(j + NB, b)

    for b in range(NB):
        j = NCHUNK - NB + b
        wait_gather(b)
        start_scatter(j, b)
        wait_scatter(b)

    plsc.subcore_barrier()
    pltpu.sync_copy(aggsh.at[pl.ds(row0, ROWS_PT)],
                    out_hbm.at[c, pl.ds(row0, ROWS_PT)])


def _tc_matmul_body(x_ref, w_ref, o_ref):
    o_ref[...] = jnp.dot(x_ref[...], w_ref[...],
                         preferred_element_type=jnp.float32)


_tc_matmul = pl.pallas_call(
    _tc_matmul_body,
    out_shape=jax.ShapeDtypeStruct((NPAD, D_HID), jnp.float32),
)


def _tc_scale2_body(p_ref, norm_ref, b1_ref, w2_ref, g2_ref):
    norm = norm_ref[...]
    agg = norm[:, None] * (p_ref[0] + p_ref[1]) + b1_ref[...]
    h1 = jnp.maximum(agg, 0.0)
    g2_ref[...] = jnp.dot(h1, w2_ref[...],
                          preferred_element_type=jnp.float32) * norm[:, None]


_tc_scale2 = pl.pallas_call(
    _tc_scale2_body,
    out_shape=jax.ShapeDtypeStruct((NPAD, D_HID), jnp.float32),
)


def _tc_scale3_body(p_ref, norm_ref, b2_ref, o_ref):
    o_ref[...] = norm_ref[...][:, None] * (p_ref[0] + p_ref[1]) + b2_ref[...]


_tc_scale3 = pl.pallas_call(
    _tc_scale3_body,
    out_shape=jax.ShapeDtypeStruct((NPAD, D_HID), jnp.float32),
)


def kernel(x, edge_index, W1, b1, W2, b2):
    x_pad = jnp.zeros((NPAD, D_FEAT), jnp.float32).at[:N_NODES].set(x)
    pad = jnp.full((EPAD - N_EDGES,), N_NODES, jnp.int32)
    src = jnp.concatenate([edge_index[0], pad]).reshape(NW, NCHUNK, 1, CHUNK)
    dst = jnp.concatenate([edge_index[1], pad]).reshape(NW, NCHUNK, 1, CHUNK)
    ei = jnp.concatenate([src, dst], axis=2)
    w2p = jnp.zeros((D_HID, D_HID), jnp.float32).at[:, :N_CLASSES].set(W2)
    b1r = b1.reshape(1, D_HID)
    b2p = jnp.zeros((1, D_HID), jnp.float32).at[0, :N_CLASSES].set(b2)

    h = _tc_matmul(x_pad, W1)
    p1, norm = _sc_fused1(h, ei)
    g2 = _tc_scale2(p1, norm, b1r, w2p)
    p2 = _sc_agg2(g2, ei)
    out = _tc_scale3(p2, norm, b2p)
    return out[:N_NODES, :N_CLASSES]


# speedup vs baseline: 1.2008x; 1.2008x over previous
import functools

import jax
import jax.numpy as jnp
from jax import lax
from jax.experimental import pallas as pl
from jax.experimental.pallas import tpu as pltpu
from jax.experimental.pallas import tpu_sc as plsc

N_NODES = 10000
N_EDGES = 320000
D_FEAT = 128
D_HID = 16
N_CLASSES = 7

NC = 2
NS = 16
NW = NC * NS
NPAD = 10240
EPW = N_EDGES // NW
CHUNK = 125
NCHUNK = EPW // CHUNK
NB = 4
ROWS_PT = NPAD // NS


def _mesh():
    return plsc.VectorSubcoreMesh(core_axis_name="c", subcore_axis_name="s")


_SC_PARAMS = pltpu.CompilerParams(use_tc_tiling_on_sc=False,
                                  needs_layout_passes=False)


@functools.partial(
    pl.kernel,
    out_type=jax.ShapeDtypeStruct((NC, NPAD), jnp.float32),
    mesh=_mesh(),
    compiler_params=_SC_PARAMS,
    scratch_types=[
        pltpu.VMEM((NCHUNK, CHUNK), jnp.int32),
        pltpu.VMEM((CHUNK,), jnp.float32),
        pltpu.VMEM((ROWS_PT,), jnp.float32),
        pltpu.VMEM_SHARED((NPAD,), jnp.float32),
        pltpu.SemaphoreType.DMA,
    ],
)
def _sc_degree(ei_hbm, out_hbm, dstb, onesb, zb, degsh, sem):
    c = lax.axis_index("c")
    s = lax.axis_index("s")
    wid = c * NS + s

    @pl.loop(0, ROWS_PT // 16)
    def _(i):
        zb[pl.ds(i * 16, 16)] = jnp.zeros((16,), jnp.float32)

    for o in (0, 16, 32, 48, 64, 80, 96, CHUNK - 16):
        onesb[pl.ds(o, 16)] = jnp.ones((16,), jnp.float32)

    pltpu.sync_copy(zb, degsh.at[pl.ds(s * ROWS_PT, ROWS_PT)])
    pltpu.sync_copy(ei_hbm.at[1, wid], dstb)
    plsc.subcore_barrier()

    @pl.loop(0, NCHUNK, step=8)
    def _(j0):
        @pl.loop(0, 8)
        def _(b):
            pltpu.async_copy(onesb, degsh.at[dstb.at[j0 + b]], sem, add=True)

        @pl.loop(0, 8)
        def _(b):
            pltpu.make_async_copy(onesb, degsh.at[dstb.at[0]], sem).wait()

    plsc.subcore_barrier()
    pltpu.sync_copy(degsh.at[pl.ds(s * ROWS_PT, ROWS_PT)],
                    out_hbm.at[c, pl.ds(s * ROWS_PT, ROWS_PT)])


@functools.partial(
    pl.kernel,
    out_type=jax.ShapeDtypeStruct((NC, NPAD, D_HID), jnp.float32),
    mesh=_mesh(),
    compiler_params=_SC_PARAMS,
    scratch_types=[
        pltpu.VMEM((NCHUNK, CHUNK), jnp.int32),
        pltpu.VMEM((NCHUNK, CHUNK), jnp.int32),
        [pltpu.VMEM((CHUNK, D_HID), jnp.float32) for _ in range(NB)],
        pltpu.VMEM((ROWS_PT, D_HID), jnp.float32),
        pltpu.VMEM_SHARED((NPAD, D_HID), jnp.float32),
        pltpu.VMEM_SHARED((NPAD, D_HID), jnp.float32),
        [pltpu.SemaphoreType.DMA for _ in range(NB)],
        [pltpu.SemaphoreType.DMA for _ in range(NB)],
    ],
)
def _sc_agg(g_hbm, ei_hbm, out_hbm, srcb, dstb, rowsb, zb, aggsh, gsh,
            gsem, ssem):
    c = lax.axis_index("c")
    s = lax.axis_index("s")
    wid = c * NS + s
    row0 = s * ROWS_PT

    @pl.loop(0, ROWS_PT)
    def _(i):
        zb[i, :] = jnp.zeros((D_HID,), jnp.float32)

    pltpu.sync_copy(zb, aggsh.at[pl.ds(row0, ROWS_PT)])
    pltpu.sync_copy(g_hbm.at[pl.ds(row0, ROWS_PT)],
                    gsh.at[pl.ds(row0, ROWS_PT)])
    pltpu.sync_copy(ei_hbm.at[0, wid], srcb)
    pltpu.sync_copy(ei_hbm.at[1, wid], dstb)
    plsc.subcore_barrier()

    def start_gather(j, b):
        pltpu.async_copy(gsh.at[srcb.at[j]], rowsb[b], gsem[b])

    def wait_gather(b):
        pltpu.make_async_copy(gsh.at[srcb.at[0]], rowsb[b], gsem[b]).wait()

    def start_scatter(j, b):
        pltpu.async_copy(rowsb[b], aggsh.at[dstb.at[j]], ssem[b], add=True)

    def wait_scatter(b):
        pltpu.make_async_copy(rowsb[b], aggsh.at[dstb.at[0]], ssem[b]).wait()

    for b in range(NB):
        start_gather(b, b)

    @pl.loop(0, (NCHUNK - NB) // NB)
    def _(gg):
        base = gg * NB
        for b in range(NB):
            j = base + b
            wait_gather(b)
            start_scatter(j, b)
            wait_scatter(b)
            start_gather(j + NB, b)

    for b in range(NB):
        j = NCHUNK - NB + b
        wait_gather(b)
        start_scatter(j, b)
        wait_scatter(b)

    plsc.subcore_barrier()
    pltpu.sync_copy(aggsh.at[pl.ds(row0, ROWS_PT)],
                    out_hbm.at[c, pl.ds(row0, ROWS_PT)])


def _tc_matmul_body(x_ref, w_ref, o_ref):
    o_ref[:N_NODES, :] = jnp.dot(x_ref[...], w_ref[...],
                                 preferred_element_type=jnp.float32)
    o_ref[N_NODES:, :] = jnp.zeros((NPAD - N_NODES, D_HID), jnp.float32)


_tc_matmul = pl.pallas_call(
    _tc_matmul_body,
    out_shape=jax.ShapeDtypeStruct((NPAD, D_HID), jnp.float32),
)


def _tc_scale1_body(h_ref, degp_ref, g_ref, norm_ref):
    deg = degp_ref[0, :] + degp_ref[1, :]
    norm = lax.rsqrt(jnp.maximum(deg, 1.0))
    norm_ref[...] = norm
    g_ref[...] = h_ref[...] * norm[:, None]


_tc_scale1 = pl.pallas_call(
    _tc_scale1_body,
    out_shape=[
        jax.ShapeDtypeStruct((NPAD, D_HID), jnp.float32),
        jax.ShapeDtypeStruct((NPAD,), jnp.float32),
    ],
)


def _tc_scale2_body(p_ref, norm_ref, b1_ref, w2_ref, g2_ref):
    norm = norm_ref[...]
    agg = norm[:, None] * (p_ref[0] + p_ref[1]) + b1_ref[...]
    h1 = jnp.maximum(agg, 0.0)
    g2_ref[...] = jnp.dot(h1, w2_ref[...],
                          preferred_element_type=jnp.float32) * norm[:, None]


_tc_scale2 = pl.pallas_call(
    _tc_scale2_body,
    out_shape=jax.ShapeDtypeStruct((NPAD, D_HID), jnp.float32),
)


def _tc_scale3_body(p_ref, norm_ref, b2_ref, o_ref):
    o_ref[...] = norm_ref[...][:, None] * (p_ref[0] + p_ref[1]) + b2_ref[...]


_tc_scale3 = pl.pallas_call(
    _tc_scale3_body,
    out_shape=jax.ShapeDtypeStruct((NPAD, D_HID), jnp.float32),
)


def kernel(x, edge_index, W1, b1, W2, b2):
    ei = edge_index.reshape(2, NW, NCHUNK, CHUNK)
    w2p = jnp.zeros((D_HID, D_HID), jnp.float32).at[:, :N_CLASSES].set(W2)
    b1r = b1.reshape(1, D_HID)
    b2p = jnp.zeros((1, D_HID), jnp.float32).at[0, :N_CLASSES].set(b2)

    h = _tc_matmul(x, W1)
    degp = _sc_degree(ei)
    g1, norm = _tc_scale1(h, degp)
    p1 = _sc_agg(g1, ei)
    g2 = _tc_scale2(p1, norm, b1r, w2p)
    p2 = _sc_agg(g2, ei)
    out = _tc_scale3(p2, norm, b2p)
    return out[:N_NODES, :N_CLASSES]


# speedup vs baseline: 1.5242x; 1.2694x over previous
import functools

import jax
import jax.numpy as jnp
from jax import lax
from jax.experimental import pallas as pl
from jax.experimental.pallas import tpu as pltpu
from jax.experimental.pallas import tpu_sc as plsc

N_NODES = 10000
N_EDGES = 320000
D_FEAT = 128
D_HID = 16
N_CLASSES = 7

NC = 2
NS = 16
NW = NC * NS
NPAD = 10240
PK = 8
PROWS = NPAD // PK
XROWS = N_NODES // PK
EPW = N_EDGES // NW
CHUNK = 125
NCHUNK = EPW // CHUNK
NB = 4
ROWS_PT = NPAD // NS
PROWS_PT = PROWS // NS


def _mesh():
    return plsc.VectorSubcoreMesh(core_axis_name="c", subcore_axis_name="s")


_SC_PARAMS = pltpu.CompilerParams(use_tc_tiling_on_sc=False,
                                  needs_layout_passes=False)


@functools.partial(
    pl.kernel,
    out_type=jax.ShapeDtypeStruct((NC, PROWS, 128), jnp.float32),
    mesh=_mesh(),
    compiler_params=_SC_PARAMS,
    scratch_types=[
        pltpu.VMEM((NCHUNK, CHUNK), jnp.int32),
        pltpu.VMEM((CHUNK,), jnp.float32),
        pltpu.VMEM((ROWS_PT,), jnp.float32),
        pltpu.VMEM((PROWS_PT, 128), jnp.float32),
        pltpu.VMEM_SHARED((NPAD,), jnp.float32),
        pltpu.SemaphoreType.DMA,
    ],
)
def _sc_degree(ei_hbm, out_hbm, dstb, onesb, degt, repb, degsh, sem):
    c = lax.axis_index("c")
    s = lax.axis_index("s")
    wid = c * NS + s
    row0 = s * ROWS_PT

    @pl.loop(0, ROWS_PT // 16)
    def _(i):
        degt[pl.ds(i * 16, 16)] = jnp.zeros((16,), jnp.float32)

    for o in (0, 16, 32, 48, 64, 80, 96, CHUNK - 16):
        onesb[pl.ds(o, 16)] = jnp.ones((16,), jnp.float32)

    pltpu.sync_copy(degt, degsh.at[pl.ds(row0, ROWS_PT)])
    pltpu.sync_copy(ei_hbm.at[1, wid], dstb)
    plsc.subcore_barrier()

    @pl.loop(0, NCHUNK, step=8)
    def _(j0):
        @pl.loop(0, 8)
        def _(b):
            pltpu.async_copy(onesb, degsh.at[dstb.at[j0 + b]], sem, add=True)

        @pl.loop(0, 8)
        def _(b):
            pltpu.make_async_copy(onesb, degsh.at[dstb.at[0]], sem).wait()

    plsc.subcore_barrier()

    pltpu.sync_copy(degsh.at[pl.ds(row0, ROWS_PT)], degt)

    @pl.loop(0, ROWS_PT // 16)
    def _(ii):
        d16 = degt[pl.ds(ii * 16, 16)]
        for k in range(16):
            repb[ii * 2 + k // PK, pl.ds((k % PK) * 16, 16)] = (
                jnp.zeros((16,), jnp.float32) + d16[k])

    pltpu.sync_copy(repb, out_hbm.at[c, pl.ds(s * PROWS_PT, PROWS_PT)])


@functools.partial(
    pl.kernel,
    out_type=jax.ShapeDtypeStruct((NC, PROWS, 128), jnp.float32),
    mesh=_mesh(),
    compiler_params=_SC_PARAMS,
    scratch_types=[
        pltpu.VMEM((NCHUNK, CHUNK), jnp.int32),
        pltpu.VMEM((NCHUNK, CHUNK), jnp.int32),
        [pltpu.VMEM((CHUNK, D_HID), jnp.float32) for _ in range(NB)],
        pltpu.VMEM((PROWS_PT, 128), jnp.float32),
        pltpu.VMEM((ROWS_PT, D_HID), jnp.float32),
        pltpu.VMEM_SHARED((NPAD, D_HID), jnp.float32),
        pltpu.VMEM_SHARED((NPAD, D_HID), jnp.float32),
        [pltpu.SemaphoreType.DMA for _ in range(NB)],
        [pltpu.SemaphoreType.DMA for _ in range(NB)],
    ],
)
def _sc_agg(g_hbm, ei_hbm, out_hbm, srcb, dstb, rowsb, pbuf, ubuf,
            aggsh, gsh, gsem, ssem):
    c = lax.axis_index("c")
    s = lax.axis_index("s")
    wid = c * NS + s
    row0 = s * ROWS_PT
    prow0 = s * PROWS_PT

    pltpu.sync_copy(g_hbm.at[pl.ds(prow0, PROWS_PT)], pbuf)
    pltpu.sync_copy(ei_hbm.at[0, wid], srcb)
    pltpu.sync_copy(ei_hbm.at[1, wid], dstb)

    @pl.loop(0, PROWS_PT)
    def _(r):
        for k in range(PK):
            ubuf[r * PK + k, :] = pbuf[r, pl.ds(k * 16, 16)]

    pltpu.sync_copy(ubuf, gsh.at[pl.ds(row0, ROWS_PT)])

    @pl.loop(0, ROWS_PT)
    def _(i):
        ubuf[i, :] = jnp.zeros((D_HID,), jnp.float32)

    pltpu.sync_copy(ubuf, aggsh.at[pl.ds(row0, ROWS_PT)])
    plsc.subcore_barrier()

    def start_gather(j, b):
        pltpu.async_copy(gsh.at[srcb.at[j]], rowsb[b], gsem[b])

    def wait_gather(b):
        pltpu.make_async_copy(gsh.at[srcb.at[0]], rowsb[b], gsem[b]).wait()

    def start_scatter(j, b):
        pltpu.async_copy(rowsb[b], aggsh.at[dstb.at[j]], ssem[b], add=True)

    def wait_scatter(b):
        pltpu.make_async_copy(rowsb[b], aggsh.at[dstb.at[0]], ssem[b]).wait()

    for b in range(NB):
        start_gather(b, b)

    @pl.loop(0, (NCHUNK - NB) // NB)
    def _(gg):
        base = gg * NB
        for b in range(NB):
            j = base + b
            wait_gather(b)
            start_scatter(j, b)
            wait_scatter(b)
            start_gather(j + NB, b)

    for b in range(NB):
        j = NCHUNK - NB + b
        wait_gather(b)
        start_scatter(j, b)
        wait_scatter(b)

    plsc.subcore_barrier()

    pltpu.sync_copy(aggsh.at[pl.ds(row0, ROWS_PT)], ubuf)

    @pl.loop(0, PROWS_PT)
    def _(r):
        for k in range(PK):
            pbuf[r, pl.ds(k * 16, 16)] = ubuf[r * PK + k, :]

    pltpu.sync_copy(pbuf, out_hbm.at[c, pl.ds(prow0, PROWS_PT)])


def _tc_matmul_body(x_ref, w_ref, o_ref):
    o_ref[:XROWS, :] = jnp.dot(x_ref[...], w_ref[...],
                               preferred_element_type=jnp.float32)
    o_ref[XROWS:, :] = jnp.zeros((PROWS - XROWS, 128), jnp.float32)


_tc_matmul = pl.pallas_call(
    _tc_matmul_body,
    out_shape=jax.ShapeDtypeStruct((PROWS, 128), jnp.float32),
)


def _tc_scale1_body(h_ref, degp_ref, g_ref, norm_ref):
    deg = degp_ref[0] + degp_ref[1]
    norm = lax.rsqrt(jnp.maximum(deg, 1.0))
    norm_ref[...] = norm
    g_ref[...] = h_ref[...] * norm


_tc_scale1 = pl.pallas_call(
    _tc_scale1_body,
    out_shape=[
        jax.ShapeDtypeStruct((PROWS, 128), jnp.float32),
        jax.ShapeDtypeStruct((PROWS, 128), jnp.float32),
    ],
)


def _tc_scale2_body(p_ref, norm_ref, b1_ref, w2bd_ref, g2_ref):
    norm = norm_ref[...]
    h1 = jnp.maximum(norm * (p_ref[0] + p_ref[1]) + b1_ref[...], 0.0)
    g2_ref[...] = jnp.dot(h1, w2bd_ref[...],
                          preferred_element_type=jnp.float32) * norm


_tc_scale2 = pl.pallas_call(
    _tc_scale2_body,
    out_shape=jax.ShapeDtypeStruct((PROWS, 128), jnp.float32),
)


def _tc_scale3_body(p_ref, norm_ref, b2_ref, o_ref):
    o_ref[...] = norm_ref[...] * (p_ref[0] + p_ref[1]) + b2_ref[...]


_tc_scale3 = pl.pallas_call(
    _tc_scale3_body,
    out_shape=jax.ShapeDtypeStruct((PROWS, 128), jnp.float32),
)


def kernel(x, edge_index, W1, b1, W2, b2):
    ei = edge_index.reshape(2, NW, NCHUNK, CHUNK)
    x2 = x.reshape(XROWS, PK * D_FEAT)
    eye8 = jnp.eye(PK, dtype=jnp.float32)
    w1bd = jnp.kron(eye8, W1)
    w2p = jnp.zeros((D_HID, D_HID), jnp.float32).at[:, :N_CLASSES].set(W2)
    w2bd = jnp.kron(eye8, w2p)
    b1t = jnp.tile(b1, PK).reshape(1, 128)
    b2p = jnp.zeros((D_HID,), jnp.float32).at[:N_CLASSES].set(b2)
    b2t = jnp.tile(b2p, PK).reshape(1, 128)

    h = _tc_matmul(x2, w1bd)
    degp = _sc_degree(ei)
    g1, norm = _tc_scale1(h, degp)
    p1 = _sc_agg(g1, ei)
    g2 = _tc_scale2(p1, norm, b1t, w2bd)
    p2 = _sc_agg(g2, ei)
    out = _tc_scale3(p2, norm, b2t)
    return out.reshape(NPAD, D_HID)[:N_NODES, :N_CLASSES]
